# baseline (speedup 1.0000x reference)
import jax
import jax.numpy as jnp
from jax import lax
from jax.experimental import pallas as pl
from jax.experimental.pallas import tpu as pltpu, tpu_sc as plsc

MAX_LEN = 512
EMBED = 64
BATCH = 1024
SEQ = 512

N = BATCH * SEQ
NC, NS = 2, 16
NW = NC * NS
ROWS_PER_W = N // NW
CHUNK = 128
CHUNKS = ROWS_PER_W // CHUNK
POSW = MAX_LEN // CHUNK
NBUF = 8


def _body(x_hbm, tok_hbm, pos_hbm, out_hbm, pos_sh, idx_v, rows,
          isems, gsems, osems):
    cid = lax.axis_index("c")
    sid = lax.axis_index("s")
    wid = sid * NC + cid
    base_w = wid * ROWS_PER_W

    @pl.when(sid == 0)
    def _():
        pltpu.sync_copy(pos_hbm, pos_sh)

    plsc.subcore_barrier()

    pltpu.sync_copy(x_hbm.at[pl.ds(wid * CHUNKS, CHUNKS)], idx_v)

    def pos_src(b):
        return pos_sh.at[pl.ds((b % POSW) * CHUNK, CHUNK)]

    def issue_init(b):
        pltpu.async_copy(pos_src(b), rows[b], isems[b])

    def issue_gather(c, b):
        pltpu.make_async_copy(pos_src(b), rows[b], isems[b]).wait()
        pltpu.async_copy(tok_hbm.at[idx_v.at[c]], rows[b], gsems[b], add=True)

    def issue_scatter(c, b):
        pltpu.make_async_copy(tok_hbm.at[idx_v.at[0]], rows[b], gsems[b]).wait()
        pltpu.async_copy(rows[b], out_hbm.at[pl.ds(base_w + c * CHUNK, CHUNK)],
                         osems[b])

    def wait_scatter(b):
        pltpu.make_async_copy(rows[b], out_hbm.at[pl.ds(base_w, CHUNK)],
                              osems[b]).wait()

    for c in range(NBUF):
        issue_init(c)
        if c >= 2:
            issue_gather(c - 2, c - 2)
        if c >= 4:
            issue_scatter(c - 4, c - 4)

    @pl.loop(NBUF, CHUNKS, step=NBUF)
    def _(go):
        for b in range(NBUF):
            c = go + b
            wait_scatter(b)
            issue_init(b)
            b2 = (b + NBUF - 2) % NBUF
            issue_gather(c - 2, b2)
            b4 = (b + NBUF - 4) % NBUF
            issue_scatter(c - 4, b4)

    for c in range(CHUNKS, CHUNKS + 4):
        if c - 2 < CHUNKS:
            issue_gather(c - 2, (c - 2) % NBUF)
        issue_scatter(c - 4, (c - 4) % NBUF)
    for b in range(NBUF):
        wait_scatter(b)


def kernel(x, token_table, pos_table):
    xf = x.reshape(NW * CHUNKS, CHUNK)
    mesh = plsc.VectorSubcoreMesh(
        core_axis_name="c", subcore_axis_name="s", num_cores=NC, num_subcores=NS
    )

    def body(x_ref, tok_ref, pos_ref, out_ref, pos_sh, idx_v, *rest):
        rows = list(rest[:NBUF])
        isems = list(rest[NBUF:2 * NBUF])
        gsems = list(rest[2 * NBUF:3 * NBUF])
        osems = list(rest[3 * NBUF:4 * NBUF])
        _body(x_ref, tok_ref, pos_ref, out_ref, pos_sh, idx_v, rows,
              isems, gsems, osems)

    run = pl.kernel(
        body,
        out_type=jax.ShapeDtypeStruct((N, EMBED), jnp.float32),
        mesh=mesh,
        scratch_types=[
            pltpu.VMEM_SHARED((MAX_LEN, EMBED), jnp.float32),
            pltpu.VMEM((CHUNKS, CHUNK), jnp.int32),
        ] + [pltpu.VMEM((CHUNK, EMBED), jnp.float32) for _ in range(NBUF)]
          + [pltpu.SemaphoreType.DMA for _ in range(3 * NBUF)],
        compiler_params=pltpu.CompilerParams(
            use_tc_tiling_on_sc=False, needs_layout_passes=False
        ),
    )
    out = run(xf, token_table, pos_table)
    return out.reshape(BATCH, SEQ, EMBED)


# speedup vs baseline: 1.1198x; 1.1198x over previous
import jax
import jax.numpy as jnp
from jax import lax
from jax.experimental import pallas as pl
from jax.experimental.pallas import tpu as pltpu, tpu_sc as plsc

MAX_LEN = 512
EMBED = 64
BATCH = 1024
SEQ = 512

N = BATCH * SEQ
NC, NS = 2, 16
NW = NC * NS
ROWS_PER_W = N // NW
CHUNK = 128
CHUNKS = ROWS_PER_W // CHUNK
POSW = MAX_LEN // CHUNK
LANES = 16
NBUF = 4
BATCH_PER_W = ROWS_PER_W // SEQ
TR = EMBED // 8


def _transpose(src, dst, iota, cols, trs, sls):
    @pl.loop(0, CHUNK // LANES, unroll=2)
    def _(k):
        rowvec = iota + k * LANES
        for i in range(EMBED):
            v = plsc.load_gather(src, [rowvec, cols[i]])
            plsc.store_scatter(dst, [trs[i], sls[i], rowvec], v)


def _body(x_hbm, tok_hbm, pos_hbm, out_hbm, pos_sh, idx_v, srcs, dsts,
          isems, gsems, osems):
    cid = lax.axis_index("c")
    sid = lax.axis_index("s")
    wid = sid * NC + cid
    batch_w = wid * BATCH_PER_W

    iota = lax.iota(jnp.int32, LANES)
    cols, trs, sls = [], [], []
    for eq in range(EMBED // LANES):
        for d in range(LANES):
            col = eq * LANES + lax.rem(iota + d, LANES)
            cols.append(col)
            trs.append(col // 8)
            sls.append(lax.rem(col, 8))

    @pl.when(sid == 0)
    def _():
        pltpu.sync_copy(pos_hbm, pos_sh)

    plsc.subcore_barrier()

    pltpu.sync_copy(x_hbm.at[pl.ds(wid * CHUNKS, CHUNKS)], idx_v)

    def issue_init(b):
        pltpu.async_copy(pos_sh.at[pl.ds(b * CHUNK, CHUNK)], srcs[b],
                         isems[b])

    def issue_gather(c, b):
        pltpu.make_async_copy(pos_sh.at[pl.ds(0, CHUNK)], srcs[b],
                              isems[b]).wait()
        pltpu.async_copy(tok_hbm.at[idx_v.at[c]], srcs[b], gsems[b], add=True)

    def finish_chunk(c, b):
        pltpu.make_async_copy(tok_hbm.at[idx_v.at[0]], srcs[b], gsems[b]).wait()
        _transpose(srcs[b], dsts[b], iota, cols, trs, sls)
        batch = batch_w + c // POSW
        pltpu.async_copy(dsts[b], out_hbm.at[batch, :, b, :, :], osems[b])

    def wait_scatter(b):
        pltpu.make_async_copy(dsts[b], out_hbm.at[0, :, b, :, :],
                              osems[b]).wait()

    for c in range(NBUF):
        issue_init(c)
        if c >= 1:
            issue_gather(c - 1, c - 1)
        if c >= 2:
            finish_chunk(c - 2, c - 2)

    @pl.loop(NBUF, CHUNKS, step=NBUF)
    def _(go):
        for b in range(NBUF):
            c = go + b
            wait_scatter(b)
            issue_init(b)
            b1 = (b + NBUF - 1) % NBUF
            issue_gather(c - 1, b1)
            b2 = (b + NBUF - 2) % NBUF
            finish_chunk(c - 2, b2)

    issue_gather(CHUNKS - 1, (CHUNKS - 1) % NBUF)
    finish_chunk(CHUNKS - 2, (CHUNKS - 2) % NBUF)
    finish_chunk(CHUNKS - 1, (CHUNKS - 1) % NBUF)
    for b in range(NBUF):
        wait_scatter(b)


def kernel(x, token_table, pos_table):
    xf = x.reshape(NW * CHUNKS, CHUNK)
    mesh = plsc.VectorSubcoreMesh(
        core_axis_name="c", subcore_axis_name="s", num_cores=NC, num_subcores=NS
    )

    def body(x_ref, tok_ref, pos_ref, out_ref, pos_sh, idx_v, *rest):
        srcs = list(rest[:NBUF])
        dsts = list(rest[NBUF:2 * NBUF])
        isems = list(rest[2 * NBUF:3 * NBUF])
        gsems = list(rest[3 * NBUF:4 * NBUF])
        osems = list(rest[4 * NBUF:5 * NBUF])
        _body(x_ref, tok_ref, pos_ref, out_ref, pos_sh, idx_v, srcs, dsts,
              isems, gsems, osems)

    run = pl.kernel(
        body,
        out_type=jax.ShapeDtypeStruct((BATCH, TR, POSW, 8, 128), jnp.float32),
        mesh=mesh,
        scratch_types=[
            pltpu.VMEM_SHARED((MAX_LEN, EMBED), jnp.float32),
            pltpu.VMEM((CHUNKS, CHUNK), jnp.int32),
        ] + [pltpu.VMEM((CHUNK, EMBED), jnp.float32) for _ in range(NBUF)]
          + [pltpu.VMEM((TR, 8, 128), jnp.float32) for _ in range(NBUF)]
          + [pltpu.SemaphoreType.DMA for _ in range(3 * NBUF)],
        compiler_params=pltpu.CompilerParams(
            use_tc_tiling_on_sc=False, needs_layout_passes=False
        ),
    )
    out = run(xf, token_table, pos_table)
    return out.transpose(0, 2, 4, 1, 3).reshape(BATCH, SEQ, EMBED)
